# baseline (speedup 1.0000x reference)
import jax
import jax.numpy as jnp
from jax import lax
from jax.experimental import pallas as pl
from jax.experimental.pallas import tpu as pltpu
from jax.experimental.pallas import tpu_sc as plsc

N = 10000
E = 320000
D = 128

NC = 2
NS = 16
NW = NC * NS
C = 128
PAIR = 2
CE = PAIR * C
E_PER_W = E // NW
NROW = 80
NHALF = NROW // 2
E_PAD_W = NROW * C
ACC_ROWS = 10240
ROWS_PER_TILE = ACC_ROWS // NS

_mesh = plsc.VectorSubcoreMesh(core_axis_name="c", subcore_axis_name="s")


def _agg_body(h_hbm, srcp_hbm, dstp_hbm, out_hbm,
              acc, src1d, dst2d, rows, sem):
    c = lax.axis_index("c")
    s = lax.axis_index("s")
    wid = s * NC + c

    zv = jnp.zeros((16,), jnp.float32)

    def _zero_row(r, _):
        for k in range(D // 16):
            rows[r, pl.ds(k * 16, 16)] = zv
        return 0

    lax.fori_loop(0, CE, _zero_row, 0)
    base = s * ROWS_PER_TILE
    for k in range(ROWS_PER_TILE // CE):
        pltpu.sync_copy(rows, acc.at[pl.ds(base + k * CE, CE)])
    rem = ROWS_PER_TILE % CE
    if rem:
        pltpu.sync_copy(rows.at[pl.ds(0, rem)],
                        acc.at[pl.ds(base + ROWS_PER_TILE - rem, rem)])

    plsc.subcore_barrier()

    for half in range(NROW // NHALF):
        pltpu.sync_copy(srcp_hbm.at[wid, pl.ds(half * NHALF * C, NHALF * C)],
                        src1d)
        pltpu.sync_copy(dstp_hbm.at[wid, pl.ds(half * NHALF, NHALF)], dst2d)

        def _chunk(i, _):
            pltpu.async_copy(
                h_hbm.at[src1d.at[pl.ds(i * CE, CE)]], rows, sem).wait()
            for b in range(PAIR):
                pltpu.sync_copy(rows.at[pl.ds(b * C, C)],
                                acc.at[dst2d.at[PAIR * i + b]], add=True)
            return 0

        lax.fori_loop(0, NHALF // PAIR, _chunk, 0)

    plsc.subcore_barrier()

    r0 = s * ROWS_PER_TILE
    pltpu.sync_copy(acc.at[pl.ds(r0, ROWS_PER_TILE)],
                    out_hbm.at[c, pl.ds(r0, ROWS_PER_TILE)])


_agg = pl.kernel(
    _agg_body,
    out_type=jax.ShapeDtypeStruct((NC, ACC_ROWS, D), jnp.float32),
    mesh=_mesh,
    scratch_types=[
        pltpu.VMEM_SHARED((ACC_ROWS, D), jnp.float32),
        pltpu.VMEM((NHALF * C,), jnp.int32),
        pltpu.VMEM((NHALF, C), jnp.int32),
        pltpu.VMEM((CE, D), jnp.float32),
        pltpu.SemaphoreType.DMA,
    ],
)


_RB = 1000


def _mm_body(x_ref, w_ref, o_ref):
    o_ref[...] = jnp.dot(x_ref[...], w_ref[...],
                         preferred_element_type=jnp.float32)


def _mm(x, w):
    return pl.pallas_call(
        _mm_body,
        grid=(N // _RB,),
        in_specs=[
            pl.BlockSpec((_RB, D), lambda i: (i, 0)),
            pl.BlockSpec((D, D), lambda i: (0, 0)),
        ],
        out_specs=pl.BlockSpec((_RB, D), lambda i: (i, 0)),
        out_shape=jax.ShapeDtypeStruct((N, D), jnp.float32),
    )(x, w)


def _combine_mm_body(p_ref, b_ref, w_ref, o_ref):
    h = p_ref[0] + p_ref[1] + b_ref[...]
    h = jnp.maximum(h, 0.0)
    o_ref[...] = jnp.dot(h, w_ref[...], preferred_element_type=jnp.float32)


def _combine_mm(p, b, w):
    return pl.pallas_call(
        _combine_mm_body,
        grid=(N // _RB,),
        in_specs=[
            pl.BlockSpec((NC, _RB, D), lambda i: (0, i, 0)),
            pl.BlockSpec((1, D), lambda i: (0, 0)),
            pl.BlockSpec((D, D), lambda i: (0, 0)),
        ],
        out_specs=pl.BlockSpec((_RB, D), lambda i: (i, 0)),
        out_shape=jax.ShapeDtypeStruct((N, D), jnp.float32),
    )(p, b.reshape(1, D), w)


def _final_body(p_ref, b_ref, o_ref):
    h = p_ref[0] + p_ref[1] + b_ref[...]
    m = jnp.max(h, axis=-1, keepdims=True)
    sh = h - m
    o_ref[...] = sh - jnp.log(jnp.sum(jnp.exp(sh), axis=-1, keepdims=True))


def _final(p, b):
    return pl.pallas_call(
        _final_body,
        grid=(N // _RB,),
        in_specs=[
            pl.BlockSpec((NC, _RB, D), lambda i: (0, i, 0)),
            pl.BlockSpec((1, D), lambda i: (0, 0)),
        ],
        out_specs=pl.BlockSpec((_RB, D), lambda i: (i, 0)),
        out_shape=jax.ShapeDtypeStruct((N, D), jnp.float32),
    )(p, b.reshape(1, D))


def kernel(x, edge_index, W1, b1, W2, b2, W3, b3):
    src = edge_index[0].reshape(NW, E_PER_W)
    dst = edge_index[1].reshape(NW, E_PER_W)
    pad = E_PAD_W - E_PER_W
    srcp = jnp.concatenate(
        [src, jnp.zeros((NW, pad), jnp.int32)], axis=1)
    dstp = jnp.concatenate(
        [dst, jnp.full((NW, pad), N, jnp.int32)], axis=1).reshape(NW, NROW, C)

    h = _mm(x, W1)
    p = _agg(h, srcp, dstp)
    h = _combine_mm(p, b1, W2)
    p = _agg(h, srcp, dstp)
    h = _combine_mm(p, b2, W3)
    p = _agg(h, srcp, dstp)
    return _final(p, b3)


# speedup vs baseline: 1.1024x; 1.1024x over previous
import functools

import jax
import jax.numpy as jnp
from jax import lax
from jax.experimental import pallas as pl
from jax.experimental.pallas import tpu as pltpu
from jax.experimental.pallas import tpu_sc as plsc

N = 10000
E = 320000
D = 128

NC = 2
NS = 16
NW = NC * NS
C = 128
E_PER_W = E // NW
NCHUNK = 80
NHALF = NCHUNK // 2
E_PAD_W = NCHUNK * C
ACC_ROWS = 10240
ROWS_PER_TILE = ACC_ROWS // NS
OUT_ROWS_PER_TILE = N // NS

_mesh = plsc.VectorSubcoreMesh(core_axis_name="c", subcore_axis_name="s")


def _agg_body(h_hbm, srcp_hbm, dstp_hbm, out_hbm,
              acc, src2d, dst2d, rows0, rows1, sem0, sem1):
    c = lax.axis_index("c")
    s = lax.axis_index("s")
    wid = s * NC + c

    zv = jnp.zeros((16,), jnp.float32)

    def _zero_row(r, _):
        for k in range(D // 16):
            rows0[r, pl.ds(k * 16, 16)] = zv
        return 0

    lax.fori_loop(0, C, _zero_row, 0)
    base = s * ROWS_PER_TILE
    for k in range(ROWS_PER_TILE // C):
        pltpu.sync_copy(rows0, acc.at[pl.ds(base + k * C, C)])

    plsc.subcore_barrier()

    for half in range(NCHUNK // NHALF):
        pltpu.sync_copy(srcp_hbm.at[wid, pl.ds(half * NHALF, NHALF)], src2d)
        pltpu.sync_copy(dstp_hbm.at[wid, pl.ds(half * NHALF, NHALF)], dst2d)
        pltpu.async_copy(h_hbm.at[src2d.at[0]], rows0, sem0)

        def _chunk(i, _):
            j0 = 2 * i
            pltpu.async_copy(h_hbm.at[src2d.at[j0 + 1]], rows1, sem1)
            pltpu.make_async_copy(h_hbm.at[pl.ds(0, C)], rows0, sem0).wait()
            pltpu.sync_copy(rows0, acc.at[dst2d.at[j0]], add=True)

            @pl.when(i < NHALF // 2 - 1)
            def _():
                pltpu.async_copy(h_hbm.at[src2d.at[j0 + 2]], rows0, sem0)

            pltpu.make_async_copy(h_hbm.at[pl.ds(0, C)], rows1, sem1).wait()
            pltpu.sync_copy(rows1, acc.at[dst2d.at[j0 + 1]], add=True)
            return 0

        lax.fori_loop(0, NHALF // 2, _chunk, 0)

    plsc.subcore_barrier()

    r0 = s * ROWS_PER_TILE
    pltpu.sync_copy(acc.at[pl.ds(r0, ROWS_PER_TILE)],
                    out_hbm.at[c, pl.ds(r0, ROWS_PER_TILE)])


_agg = pl.kernel(
    _agg_body,
    out_type=jax.ShapeDtypeStruct((NC, ACC_ROWS, D), jnp.float32),
    mesh=_mesh,
    scratch_types=[
        pltpu.VMEM_SHARED((ACC_ROWS, D), jnp.float32),
        pltpu.VMEM((NHALF, C), jnp.int32),
        pltpu.VMEM((NHALF, C), jnp.int32),
        pltpu.VMEM((C, D), jnp.float32),
        pltpu.VMEM((C, D), jnp.float32),
        pltpu.SemaphoreType.DMA,
        pltpu.SemaphoreType.DMA,
    ],
)


_RB = 1000


def _mm_body(x_ref, w_ref, o_ref):
    o_ref[...] = jnp.dot(x_ref[...], w_ref[...],
                         preferred_element_type=jnp.float32)


def _mm(x, w):
    return pl.pallas_call(
        _mm_body,
        grid=(N // _RB,),
        in_specs=[
            pl.BlockSpec((_RB, D), lambda i: (i, 0)),
            pl.BlockSpec((D, D), lambda i: (0, 0)),
        ],
        out_specs=pl.BlockSpec((_RB, D), lambda i: (i, 0)),
        out_shape=jax.ShapeDtypeStruct((N, D), jnp.float32),
    )(x, w)


def _combine_mm_body(p_ref, b_ref, w_ref, o_ref):
    h = p_ref[0] + p_ref[1] + b_ref[...]
    h = jnp.maximum(h, 0.0)
    o_ref[...] = jnp.dot(h, w_ref[...], preferred_element_type=jnp.float32)


def _combine_mm(p, b, w):
    return pl.pallas_call(
        _combine_mm_body,
        grid=(N // _RB,),
        in_specs=[
            pl.BlockSpec((NC, _RB, D), lambda i: (0, i, 0)),
            pl.BlockSpec((1, D), lambda i: (0, 0)),
            pl.BlockSpec((D, D), lambda i: (0, 0)),
        ],
        out_specs=pl.BlockSpec((_RB, D), lambda i: (i, 0)),
        out_shape=jax.ShapeDtypeStruct((N, D), jnp.float32),
    )(p, b.reshape(1, D), w)


def _final_body(p_ref, b_ref, o_ref):
    h = p_ref[0] + p_ref[1] + b_ref[...]
    m = jnp.max(h, axis=-1, keepdims=True)
    sh = h - m
    o_ref[...] = sh - jnp.log(jnp.sum(jnp.exp(sh), axis=-1, keepdims=True))


def _final(p, b):
    return pl.pallas_call(
        _final_body,
        grid=(N // _RB,),
        in_specs=[
            pl.BlockSpec((NC, _RB, D), lambda i: (0, i, 0)),
            pl.BlockSpec((1, D), lambda i: (0, 0)),
        ],
        out_specs=pl.BlockSpec((_RB, D), lambda i: (i, 0)),
        out_shape=jax.ShapeDtypeStruct((N, D), jnp.float32),
    )(p, b.reshape(1, D))


def kernel(x, edge_index, W1, b1, W2, b2, W3, b3):
    src = edge_index[0].reshape(NW, E_PER_W)
    dst = edge_index[1].reshape(NW, E_PER_W)
    pad = E_PAD_W - E_PER_W
    srcp = jnp.concatenate(
        [src, jnp.zeros((NW, pad), jnp.int32)], axis=1).reshape(NW, NCHUNK, C)
    dstp = jnp.concatenate(
        [dst, jnp.full((NW, pad), N, jnp.int32)], axis=1).reshape(NW, NCHUNK, C)

    h = _mm(x, W1)
    p = _agg(h, srcp, dstp)
    h = _combine_mm(p, b1, W2)
    p = _agg(h, srcp, dstp)
    h = _combine_mm(p, b2, W3)
    p = _agg(h, srcp, dstp)
    return _final(p, b3)


# speedup vs baseline: 1.1037x; 1.0012x over previous
import jax
import jax.numpy as jnp
from jax import lax
from jax.experimental import pallas as pl
from jax.experimental.pallas import tpu as pltpu
from jax.experimental.pallas import tpu_sc as plsc

N = 10000
E = 320000
D = 128

NC = 2
NS = 16
NW = NC * NS
C = 128
E_PER_W = E // NW
NCHUNK = 80
NHALF = NCHUNK // 2
E_PAD_W = NCHUNK * C
ACC_ROWS = 10240
ROWS_PER_TILE = ACC_ROWS // NS

_mesh = plsc.VectorSubcoreMesh(core_axis_name="c", subcore_axis_name="s")


def _agg_body(h_hbm, srcp_hbm, dstp_hbm, out_hbm,
              acc, src2d, dst2d, rows0, rows1, sem0, sem1):
    c = lax.axis_index("c")
    s = lax.axis_index("s")
    wid = s * NC + c

    zv = jnp.zeros((16,), jnp.float32)

    def _zero_row(r, _):
        for k in range(D // 16):
            rows0[r, pl.ds(k * 16, 16)] = zv
        return 0

    lax.fori_loop(0, C, _zero_row, 0)
    base = s * ROWS_PER_TILE
    for k in range(ROWS_PER_TILE // C):
        pltpu.sync_copy(rows0, acc.at[pl.ds(base + k * C, C)])

    plsc.subcore_barrier()

    for half in range(NCHUNK // NHALF):
        pltpu.sync_copy(srcp_hbm.at[wid, pl.ds(half * NHALF, NHALF)], src2d)
        pltpu.sync_copy(dstp_hbm.at[wid, pl.ds(half * NHALF, NHALF)], dst2d)
        pltpu.async_copy(h_hbm.at[src2d.at[0]], rows0, sem0)

        def _chunk(i, _):
            j0 = 2 * i
            pltpu.async_copy(h_hbm.at[src2d.at[j0 + 1]], rows1, sem1)
            pltpu.make_async_copy(h_hbm.at[pl.ds(0, C)], rows0, sem0).wait()
            pltpu.sync_copy(rows0, acc.at[dst2d.at[j0]], add=True)

            @pl.when(i < NHALF // 2 - 1)
            def _():
                pltpu.async_copy(h_hbm.at[src2d.at[j0 + 2]], rows0, sem0)

            pltpu.make_async_copy(h_hbm.at[pl.ds(0, C)], rows1, sem1).wait()
            pltpu.sync_copy(rows1, acc.at[dst2d.at[j0 + 1]], add=True)
            return 0

        lax.fori_loop(0, NHALF // 2, _chunk, 0)

    plsc.subcore_barrier()

    r0 = s * ROWS_PER_TILE
    pltpu.sync_copy(acc.at[pl.ds(r0, ROWS_PER_TILE)],
                    out_hbm.at[c, pl.ds(r0, ROWS_PER_TILE)])


_agg = pl.kernel(
    _agg_body,
    out_type=jax.ShapeDtypeStruct((NC, ACC_ROWS, D), jnp.float32),
    mesh=_mesh,
    scratch_types=[
        pltpu.VMEM_SHARED((ACC_ROWS, D), jnp.float32),
        pltpu.VMEM((NHALF, C), jnp.int32),
        pltpu.VMEM((NHALF, C), jnp.int32),
        pltpu.VMEM((C, D), jnp.float32),
        pltpu.VMEM((C, D), jnp.float32),
        pltpu.SemaphoreType.DMA,
        pltpu.SemaphoreType.DMA,
    ],
)


_RB = 1000


def _mm_body(x_ref, w_ref, o_ref):
    o_ref[...] = jnp.dot(x_ref[...], w_ref[...],
                         preferred_element_type=jnp.float32)


def _mm(x, w):
    return pl.pallas_call(
        _mm_body,
        grid=(N // _RB,),
        in_specs=[
            pl.BlockSpec((_RB, D), lambda i: (i, 0)),
            pl.BlockSpec((D, D), lambda i: (0, 0)),
        ],
        out_specs=pl.BlockSpec((_RB, D), lambda i: (i, 0)),
        out_shape=jax.ShapeDtypeStruct((N, D), jnp.float32),
    )(x, w)


def _combine_mm_body(p_ref, b_ref, w_ref, o_ref):
    h = p_ref[0] + p_ref[1] + b_ref[...]
    h = jnp.maximum(h, 0.0)
    o_ref[...] = jnp.dot(h, w_ref[...], preferred_element_type=jnp.float32)


def _combine_mm(p, b, w):
    return pl.pallas_call(
        _combine_mm_body,
        grid=(N // _RB,),
        in_specs=[
            pl.BlockSpec((NC, _RB, D), lambda i: (0, i, 0)),
            pl.BlockSpec((1, D), lambda i: (0, 0)),
            pl.BlockSpec((D, D), lambda i: (0, 0)),
        ],
        out_specs=pl.BlockSpec((_RB, D), lambda i: (i, 0)),
        out_shape=jax.ShapeDtypeStruct((N, D), jnp.float32),
    )(p, b.reshape(1, D), w)


def _final_body(p_ref, b_ref, o_ref):
    h = p_ref[0] + p_ref[1] + b_ref[...]
    m = jnp.max(h, axis=-1, keepdims=True)
    sh = h - m
    o_ref[...] = sh - jnp.log(jnp.sum(jnp.exp(sh), axis=-1, keepdims=True))


def _final(p, b):
    return pl.pallas_call(
        _final_body,
        grid=(N // _RB,),
        in_specs=[
            pl.BlockSpec((NC, _RB, D), lambda i: (0, i, 0)),
            pl.BlockSpec((1, D), lambda i: (0, 0)),
        ],
        out_specs=pl.BlockSpec((_RB, D), lambda i: (i, 0)),
        out_shape=jax.ShapeDtypeStruct((N, D), jnp.float32),
    )(p, b.reshape(1, D))


def kernel(x, edge_index, W1, b1, W2, b2, W3, b3):
    src = edge_index[0].reshape(NW, E_PER_W)
    dst = edge_index[1].reshape(NW, E_PER_W)
    pad = E_PAD_W - E_PER_W
    srcp = jnp.concatenate(
        [src, jnp.zeros((NW, pad), jnp.int32)], axis=1).reshape(NW, NCHUNK, C)
    dstp = jnp.concatenate(
        [dst, jnp.full((NW, pad), N, jnp.int32)], axis=1).reshape(NW, NCHUNK, C)

    h = _mm(x, W1)
    p = _agg(h, srcp, dstp)
    h = _combine_mm(p, b1, W2)
    p = _agg(h, srcp, dstp)
    h = _combine_mm(p, b2, W3)
    p = _agg(h, srcp, dstp)
    return _final(p, b3)
